# baseline (speedup 1.0000x reference)
import functools

import jax
import jax.numpy as jnp
from jax import lax
from jax.experimental import pallas as pl
from jax.experimental.pallas import tpu as pltpu
from jax.experimental.pallas import tpu_sc as plsc

NC = 2
NS = 16
NW = NC * NS

GROUP = 128
NBUF = 8


def _body(idx_hbm, table_hbm, out_hbm, idx_v, rows_v, gsem, ssem,
          *, gpw, rpw, steps):
    c = lax.axis_index("c")
    s = lax.axis_index("s")
    wid = c * NS + s
    gbase = wid * gpw
    rbase = wid * rpw

    pltpu.sync_copy(idx_hbm.at[pl.ds(gbase, gpw)], idx_v)

    def gather(g, j):
        return pltpu.make_async_copy(
            table_hbm.at[idx_v.at[g]], rows_v.at[j], gsem.at[j])

    def scatter(g, j):
        off = pl.multiple_of(rbase + g * GROUP, GROUP)
        return pltpu.make_async_copy(
            rows_v.at[j], out_hbm.at[pl.ds(off, GROUP)], ssem.at[j])

    for j in range(NBUF):
        gather(j, j).start()

    def loop_body(k, carry):
        g0 = k * NBUF
        for j in range(NBUF):
            g = g0 + j
            gather(g, j).wait()
            scatter(g, j).start()
        for j in range(NBUF):
            gn = g0 + NBUF + j
            scatter(g0 + j, j).wait()
            gather(gn, j).start()
        return carry

    lax.fori_loop(0, steps // NBUF - 1, loop_body, 0)

    g0 = steps - NBUF
    for j in range(NBUF):
        g = g0 + j
        gather(g, j).wait()
        scatter(g, j).start()
    for j in range(NBUF):
        scatter(g0 + j, j).wait()


def kernel(word_batch, table):
    b, sq = word_batch.shape
    n = b * sq
    d = table.shape[1]
    rpw = n // NW
    steps = rpw // GROUP
    gpw = steps

    idx = word_batch.reshape(n // GROUP, GROUP).astype(jnp.int32)

    mesh = plsc.VectorSubcoreMesh(core_axis_name="c", subcore_axis_name="s")
    halves = []
    nh = n // 2
    rpw_h = rpw // 2
    steps_h = steps // 2
    body = functools.partial(_body, gpw=steps_h, rpw=rpw_h, steps=steps_h)
    for h in range(2):
        out_h = pl.kernel(
            body,
            out_type=jax.ShapeDtypeStruct((nh, d), jnp.float32),
            mesh=mesh,
            scratch_types=[
                pltpu.VMEM((steps_h, GROUP), jnp.int32),
                pltpu.VMEM((NBUF, GROUP, d), jnp.float32),
                pltpu.SemaphoreType.DMA((NBUF,)),
                pltpu.SemaphoreType.DMA((NBUF,)),
            ],
            compiler_params=pltpu.CompilerParams(use_tc_tiling_on_sc=False),
        )(idx[h * (nh // GROUP):(h + 1) * (nh // GROUP)], table)
        halves.append(out_h.reshape(b // 2, sq, d))
    return jnp.concatenate(halves, axis=0)


# speedup vs baseline: 1.0545x; 1.0545x over previous
import functools

import jax
import jax.numpy as jnp
from jax import lax
from jax.experimental import pallas as pl
from jax.experimental.pallas import tpu as pltpu
from jax.experimental.pallas import tpu_sc as plsc

NC = 2
NS = 16
NW = NC * NS

GROUP = 128
NBUF = 8


def _body(idx_hbm, table_hbm, out_hbm, idx_v, rows_v, gsem, ssem,
          *, gpw, rpw, steps):
    c = lax.axis_index("c")
    s = lax.axis_index("s")
    wid = c * NS + s
    gbase = wid * gpw
    rbase = wid * rpw

    pltpu.sync_copy(idx_hbm.at[pl.ds(gbase, gpw)], idx_v)

    def gather(g, j):
        return pltpu.make_async_copy(
            table_hbm.at[idx_v.at[g]], rows_v.at[j], gsem.at[j])

    def scatter(g, j):
        off = pl.multiple_of(rbase + g * GROUP, GROUP)
        return pltpu.make_async_copy(
            rows_v.at[j], out_hbm.at[pl.ds(off, GROUP)], ssem.at[j])

    for j in range(NBUF):
        gather(j, j).start()

    def loop_body(k, carry):
        g0 = k * NBUF
        for j in range(NBUF):
            g = g0 + j
            gather(g, j).wait()
            scatter(g, j).start()
        for j in range(NBUF):
            gn = g0 + NBUF + j
            scatter(g0 + j, j).wait()
            gather(gn, j).start()
        return carry

    lax.fori_loop(0, steps // NBUF - 1, loop_body, 0)

    g0 = steps - NBUF
    for j in range(NBUF):
        g = g0 + j
        gather(g, j).wait()
        scatter(g, j).start()
    for j in range(NBUF):
        scatter(g0 + j, j).wait()


def kernel(word_batch, table):
    b, sq = word_batch.shape
    n = b * sq
    d = table.shape[1]
    rpw = n // NW
    steps = rpw // GROUP
    gpw = steps

    idx = word_batch.reshape(n // GROUP, GROUP).astype(jnp.int32)

    mesh = plsc.VectorSubcoreMesh(core_axis_name="c", subcore_axis_name="s")
    body = functools.partial(_body, gpw=gpw, rpw=rpw, steps=steps)
    out = pl.kernel(
        body,
        out_type=jax.ShapeDtypeStruct((n, d), jnp.float32),
        mesh=mesh,
        scratch_types=[
            pltpu.VMEM((gpw, GROUP), jnp.int32),
            pltpu.VMEM((NBUF, GROUP, d), jnp.float32),
            pltpu.SemaphoreType.DMA((NBUF,)),
            pltpu.SemaphoreType.DMA((NBUF,)),
        ],
        compiler_params=pltpu.CompilerParams(use_tc_tiling_on_sc=False),
    )(idx, table)
    return out.reshape(b, sq, d)
